# baseline (speedup 1.0000x reference)
import functools

import jax
import jax.numpy as jnp
from jax import lax
from jax.experimental import pallas as pl
from jax.experimental.pallas import tpu as pltpu
from jax.experimental.pallas import tpu_sc as plsc

VOCAB = 1000000
EMBED = 64
B = 4096
L = 200
LANES = 128

NC = 2
NS = 16
NW = NC * NS
NIDX = (B * L) // LANES
CPW = NIDX // NW


def _worker_body(tok_hbm, wE_hbm, pos_hbm, out_hbm,
                 idx_v, pos_v, g0, g1, o0, o1, gs0, gs1, ws0, ws1):
    cid = lax.axis_index("c")
    sid = lax.axis_index("s")
    wid = sid * NC + cid
    base = wid * CPW

    pltpu.sync_copy(tok_hbm.at[pl.ds(base, CPW)], idx_v)
    pltpu.sync_copy(pos_hbm, pos_v)

    gbuf = (g0, g1)
    obuf = (o0, o1)
    gsem = (gs0, gs1)
    wsem = (ws0, ws1)

    def gather_copy(cc, b):
        return pltpu.make_async_copy(wE_hbm.at[idx_v.at[cc]], gbuf[b], gsem[b])

    def write_copy(cc, b):
        return pltpu.make_async_copy(obuf[b], out_hbm.at[base + cc], wsem[b])

    def chunk_step(cc, b):
        gather_copy(cc, b).wait()

        @pl.when(cc >= 2)
        def _():
            write_copy(cc - 2, b).wait()

        start = lax.rem(cc * LANES, L)
        split = L - start

        def add_lo(j, carry):
            for k in range(EMBED // 16):
                sl = pl.ds(16 * k, 16)
                obuf[b][j, sl] = gbuf[b][j, sl] + pos_v[start + j, sl]
            return carry

        def add_hi(j, carry):
            for k in range(EMBED // 16):
                sl = pl.ds(16 * k, 16)
                obuf[b][j, sl] = gbuf[b][j, sl] + pos_v[start + j - L, sl]
            return carry

        lax.fori_loop(0, lax.min(split, LANES), add_lo, 0)
        lax.fori_loop(lax.min(split, LANES), LANES, add_hi, 0)

        @pl.when(cc + 2 < CPW)
        def _():
            gather_copy(cc + 2, b).start()

        write_copy(cc, b).start()

    gather_copy(0, 0).start()
    gather_copy(1, 1).start()

    def loop_body(i, carry):
        chunk_step(2 * i, 0)
        chunk_step(2 * i + 1, 1)
        return carry

    lax.fori_loop(0, CPW // 2, loop_body, 0)

    write_copy(CPW - 2, 0).wait()
    write_copy(CPW - 1, 1).wait()


def _sc_embed(tok, W_E_wide, W_pos):
    mesh = plsc.VectorSubcoreMesh(core_axis_name="c", subcore_axis_name="s")
    kern = functools.partial(
        pl.kernel,
        out_type=jax.ShapeDtypeStruct((NIDX, LANES, EMBED), jnp.float32),
        mesh=mesh,
        scratch_types=[
            pltpu.VMEM((CPW, LANES), jnp.int32),
            pltpu.VMEM((L, EMBED), jnp.float32),
            pltpu.VMEM((LANES, LANES), jnp.float32),
            pltpu.VMEM((LANES, LANES), jnp.float32),
            pltpu.VMEM((LANES, EMBED), jnp.float32),
            pltpu.VMEM((LANES, EMBED), jnp.float32),
            pltpu.SemaphoreType.DMA,
            pltpu.SemaphoreType.DMA,
            pltpu.SemaphoreType.DMA,
            pltpu.SemaphoreType.DMA,
        ],
        compiler_params=pltpu.CompilerParams(use_tc_tiling_on_sc=True),
    )(_worker_body)
    return kern(tok, W_E_wide, W_pos)


def kernel(tokens, W_E, W_pos):
    tok = tokens.reshape(NIDX, LANES).astype(jnp.int32)
    wide = jnp.pad(W_E, ((0, 0), (0, LANES - EMBED)))
    out = _sc_embed(tok, wide, W_pos)
    return out.reshape(B, L, EMBED)
